# baseline (speedup 1.0000x reference)
import functools
import jax
import jax.numpy as jnp
from jax import lax
from jax.experimental import pallas as pl
from jax.experimental.pallas import tpu as pltpu
from jax.experimental.pallas import tpu_sc as plsc

N = 10000
E = 320000
IN_DIM = 128
OUT_DIM = 100

NP = 10240
DP = 112
NC = 2
NS = 16
NW = NC * NS
CW = 128
CH = 80
NBUF = 1
EPW = CH * CW
EPAD = NW * EPW
RPT = NP // NS

@functools.cache
def _sc_kernels():
    mesh = plsc.VectorSubcoreMesh(
        core_axis_name="c", subcore_axis_name="s",
        num_cores=NC, num_subcores=NS)

    params = pltpu.CompilerParams(use_tc_tiling_on_sc=False)

    deg_kernel = functools.partial(
        pl.kernel,
        out_type=jax.ShapeDtypeStruct((NC, NP, 16), jnp.float32),
        mesh=mesh,
        compiler_params=params,
        scratch_types=[
            pltpu.VMEM((CH, CW), jnp.int32),
            pltpu.VMEM((CW, 16), jnp.float32),
            pltpu.VMEM_SHARED((NP, 16), jnp.float32),
            pltpu.SemaphoreType.DMA,
        ],
    )(_deg_body)

    edge_kernel = functools.partial(
        pl.kernel,
        out_type=jax.ShapeDtypeStruct((NC, NP, DP), jnp.float32),
        mesh=mesh,
        compiler_params=params,
        scratch_types=[
            pltpu.VMEM((CH, CW), jnp.int32),
            pltpu.VMEM((CH, CW), jnp.int32),
            pltpu.VMEM((NBUF, CW, DP), jnp.float32),
            pltpu.VMEM_SHARED((NP, DP), jnp.float32),
        ] + [pltpu.SemaphoreType.DMA] * NBUF,
    )(_edge_body)

    return deg_kernel, edge_kernel


def _deg_body(dst_hbm, zeros_hbm, ones_hbm, deg_out, dstv, onesv, acc, sem):
    c = lax.axis_index("c")
    s = lax.axis_index("s")
    wid = s * NC + c
    pltpu.sync_copy(zeros_hbm, acc.at[pl.ds(s * RPT, RPT)])
    pltpu.sync_copy(dst_hbm.at[wid], dstv)
    pltpu.sync_copy(ones_hbm, onesv)
    plsc.subcore_barrier()

    def body(j, carry):
        pltpu.sync_copy(onesv, acc.at[dstv.at[j]], add=True)
        return carry

    lax.fori_loop(0, CH, body, 0)
    plsc.subcore_barrier()
    pltpu.sync_copy(acc.at[pl.ds(s * RPT, RPT)],
                    deg_out.at[c, pl.ds(s * RPT, RPT)])


def _edge_body(hp_hbm, src_hbm, dst_hbm, zeros_hbm, acc_out,
               srcv, dstv, rows, acc, *sems):
    c = lax.axis_index("c")
    s = lax.axis_index("s")
    wid = s * NC + c
    pltpu.sync_copy(zeros_hbm, acc.at[pl.ds(s * RPT, RPT)])
    pltpu.sync_copy(src_hbm.at[wid], srcv)
    pltpu.sync_copy(dst_hbm.at[wid], dstv)
    plsc.subcore_barrier()

    def body(j, carry):
        pltpu.async_copy(hp_hbm.at[srcv.at[j]], rows.at[0], sems[0]).wait()
        pltpu.sync_copy(rows.at[0], acc.at[dstv.at[j]], add=True)
        return carry

    lax.fori_loop(0, CH, body, 0)
    plsc.subcore_barrier()
    pltpu.sync_copy(acc.at[pl.ds(s * RPT, RPT)],
                    acc_out.at[c, pl.ds(s * RPT, RPT)])


_BLK = 1024


def _scale_mm_body(x_ref, wt_ref, da_ref, db_ref, hp_ref, dis_ref):
    deg = da_ref[:, 0:1] + db_ref[:, 0:1] + 1.0
    dis = jnp.broadcast_to(lax.rsqrt(deg), (_BLK, DP))
    h = jnp.dot(x_ref[...], wt_ref[...], preferred_element_type=jnp.float32)
    hp_ref[...] = dis * h
    dis_ref[...] = dis


_scale_mm = pl.pallas_call(
    _scale_mm_body,
    grid=(NP // _BLK,),
    in_specs=[
        pl.BlockSpec((_BLK, IN_DIM), lambda i: (i, 0)),
        pl.BlockSpec((IN_DIM, DP), lambda i: (0, 0)),
        pl.BlockSpec((_BLK, 16), lambda i: (i, 0)),
        pl.BlockSpec((_BLK, 16), lambda i: (i, 0)),
    ],
    out_specs=[
        pl.BlockSpec((_BLK, DP), lambda i: (i, 0)),
        pl.BlockSpec((_BLK, DP), lambda i: (i, 0)),
    ],
    out_shape=[
        jax.ShapeDtypeStruct((NP, DP), jnp.float32),
        jax.ShapeDtypeStruct((NP, DP), jnp.float32),
    ],
)


def _epilogue_body(a0_ref, a1_ref, hp_ref, dis_ref, b_ref, out_ref):
    agg = a0_ref[...] + a1_ref[...] + hp_ref[...]
    out_ref[...] = dis_ref[...] * agg + b_ref[0:1, :]


_epilogue = pl.pallas_call(
    _epilogue_body,
    grid=(NP // _BLK,),
    in_specs=[
        pl.BlockSpec((_BLK, DP), lambda i: (i, 0)),
        pl.BlockSpec((_BLK, DP), lambda i: (i, 0)),
        pl.BlockSpec((_BLK, DP), lambda i: (i, 0)),
        pl.BlockSpec((_BLK, DP), lambda i: (i, 0)),
        pl.BlockSpec((8, DP), lambda i: (0, 0)),
    ],
    out_specs=pl.BlockSpec((_BLK, DP), lambda i: (i, 0)),
    out_shape=jax.ShapeDtypeStruct((NP, DP), jnp.float32),
)


@jax.jit
def kernel(x, edge_index, W, b):
    src = edge_index[0].astype(jnp.int32)
    dst = edge_index[1].astype(jnp.int32)
    padv = jnp.full((EPAD - E,), N, dtype=jnp.int32)
    src_p = jnp.concatenate([src, padv]).reshape(NW, CH, CW)
    dst_p = jnp.concatenate([dst, padv]).reshape(NW, CH, CW)

    x_p = jnp.pad(x, ((0, NP - N), (0, 0)))
    wt_p = jnp.pad(W.T, ((0, 0), (0, DP - OUT_DIM)))
    b_p = jnp.broadcast_to(jnp.pad(b, (0, DP - OUT_DIM))[None, :], (8, DP))

    zeros16 = jnp.zeros((RPT, 16), jnp.float32)
    ones16 = jnp.ones((CW, 16), jnp.float32)
    zerosDP = jnp.zeros((RPT, DP), jnp.float32)

    deg_kernel, edge_kernel = _sc_kernels()
    deg2 = deg_kernel(dst_p, zeros16, ones16)
    hp, dis = _scale_mm(x_p, wt_p, deg2[0], deg2[1])
    acc2 = edge_kernel(hp, src_p, dst_p, zerosDP)
    out = _epilogue(acc2[0], acc2[1], hp, dis, b_p)
    return out[:N, :OUT_DIM]


# speedup vs baseline: 1.6062x; 1.6062x over previous
import functools
import jax
import jax.numpy as jnp
from jax import lax
from jax.experimental import pallas as pl
from jax.experimental.pallas import tpu as pltpu
from jax.experimental.pallas import tpu_sc as plsc

N = 10000
E = 320000
IN_DIM = 128
OUT_DIM = 100

NP = 10240
DP = 112
NC = 2
NS = 16
NW = NC * NS
CW = 128
CH = 79
EPW = CH * CW
EPAD = NW * EPW
RPT = NP // NS

@functools.cache
def _sc_kernels():
    mesh = plsc.VectorSubcoreMesh(
        core_axis_name="c", subcore_axis_name="s",
        num_cores=NC, num_subcores=NS)

    params = pltpu.CompilerParams(use_tc_tiling_on_sc=False)

    deg_kernel = functools.partial(
        pl.kernel,
        out_type=jax.ShapeDtypeStruct((NC, NP, 16), jnp.float32),
        mesh=mesh,
        compiler_params=params,
        scratch_types=[
            pltpu.VMEM((CH, CW), jnp.int32),
            pltpu.VMEM((CW, 16), jnp.float32),
            pltpu.VMEM_SHARED((NP, 16), jnp.float32),
            pltpu.SemaphoreType.DMA,
        ],
    )(_deg_body)

    edge_kernel = functools.partial(
        pl.kernel,
        out_type=jax.ShapeDtypeStruct((NC, NP, DP), jnp.float32),
        mesh=mesh,
        compiler_params=params,
        scratch_types=[
            pltpu.VMEM((CH, CW), jnp.int32),
            pltpu.VMEM((CH, CW), jnp.int32),
            pltpu.VMEM((CW, DP), jnp.float32),
            pltpu.VMEM_SHARED((NP, DP), jnp.float32),
            pltpu.SemaphoreType.DMA,
        ],
    )(_edge_body)

    return deg_kernel, edge_kernel


def _deg_body(dst_hbm, zeros_hbm, ones_hbm, deg_out, dstv, onesv, acc, sem):
    c = lax.axis_index("c")
    s = lax.axis_index("s")
    wid = s * NC + c
    pltpu.sync_copy(zeros_hbm, acc.at[pl.ds(s * RPT, RPT)])
    pltpu.sync_copy(dst_hbm.at[wid], dstv)
    pltpu.sync_copy(ones_hbm, onesv)
    plsc.subcore_barrier()

    def body(j, carry):
        pltpu.sync_copy(onesv, acc.at[dstv.at[j]], add=True)
        return carry

    lax.fori_loop(0, CH, body, 0)
    plsc.subcore_barrier()
    pltpu.sync_copy(acc.at[pl.ds(s * RPT, RPT)],
                    deg_out.at[c, pl.ds(s * RPT, RPT)])


def _edge_body(hp_hbm, src_hbm, dst_hbm, zeros_hbm, acc_out,
               srcv, dstv, rows, acc, sem):
    c = lax.axis_index("c")
    s = lax.axis_index("s")
    wid = s * NC + c
    pltpu.sync_copy(zeros_hbm, acc.at[pl.ds(s * RPT, RPT)])
    pltpu.sync_copy(src_hbm.at[wid], srcv)
    pltpu.sync_copy(dst_hbm.at[wid], dstv)
    plsc.subcore_barrier()

    def body(j, carry):
        pltpu.async_copy(hp_hbm.at[srcv.at[j]], rows, sem).wait()
        pltpu.sync_copy(rows, acc.at[dstv.at[j]], add=True)
        return carry

    lax.fori_loop(0, CH, body, 0)
    plsc.subcore_barrier()
    pltpu.sync_copy(acc.at[pl.ds(s * RPT, RPT)],
                    acc_out.at[c, pl.ds(s * RPT, RPT)])


_BLK = 1024


def _scale_mm_body(x_ref, wt_ref, da_ref, db_ref, hp_ref, dis_ref):
    deg = da_ref[:, 0:1] + db_ref[:, 0:1] + 1.0
    dis = jnp.broadcast_to(lax.rsqrt(deg), (_BLK, DP))
    h = jnp.dot(x_ref[...], wt_ref[...], preferred_element_type=jnp.float32)
    hp_ref[...] = dis * h
    dis_ref[...] = dis


_scale_mm = pl.pallas_call(
    _scale_mm_body,
    grid=(NP // _BLK,),
    in_specs=[
        pl.BlockSpec((_BLK, IN_DIM), lambda i: (i, 0)),
        pl.BlockSpec((IN_DIM, DP), lambda i: (0, 0)),
        pl.BlockSpec((_BLK, 16), lambda i: (i, 0)),
        pl.BlockSpec((_BLK, 16), lambda i: (i, 0)),
    ],
    out_specs=[
        pl.BlockSpec((_BLK, DP), lambda i: (i, 0)),
        pl.BlockSpec((_BLK, DP), lambda i: (i, 0)),
    ],
    out_shape=[
        jax.ShapeDtypeStruct((NP, DP), jnp.float32),
        jax.ShapeDtypeStruct((NP, DP), jnp.float32),
    ],
)


def _epilogue_body(a0_ref, a1_ref, hp_ref, dis_ref, b_ref, out_ref):
    agg = a0_ref[...] + a1_ref[...] + hp_ref[...]
    out_ref[...] = dis_ref[...] * agg + b_ref[0:1, :]


_epilogue = pl.pallas_call(
    _epilogue_body,
    grid=(NP // _BLK,),
    in_specs=[
        pl.BlockSpec((_BLK, DP), lambda i: (i, 0)),
        pl.BlockSpec((_BLK, DP), lambda i: (i, 0)),
        pl.BlockSpec((_BLK, DP), lambda i: (i, 0)),
        pl.BlockSpec((_BLK, DP), lambda i: (i, 0)),
        pl.BlockSpec((8, DP), lambda i: (0, 0)),
    ],
    out_specs=pl.BlockSpec((_BLK, DP), lambda i: (i, 0)),
    out_shape=jax.ShapeDtypeStruct((NP, DP), jnp.float32),
)


@jax.jit
def kernel(x, edge_index, W, b):
    src = edge_index[0].astype(jnp.int32)
    dst = edge_index[1].astype(jnp.int32)
    padv = jnp.full((EPAD - E,), N, dtype=jnp.int32)
    src_p = jnp.concatenate([src, padv]).reshape(NW, CH, CW)
    dst_p = jnp.concatenate([dst, padv]).reshape(NW, CH, CW)

    x_p = jnp.pad(x, ((0, NP - N), (0, 0)))
    wt_p = jnp.pad(W.T, ((0, 0), (0, DP - OUT_DIM)))
    b_p = jnp.broadcast_to(jnp.pad(b, (0, DP - OUT_DIM))[None, :], (8, DP))

    zeros16 = jnp.zeros((RPT, 16), jnp.float32)
    ones16 = jnp.ones((CW, 16), jnp.float32)
    zerosDP = jnp.zeros((RPT, DP), jnp.float32)

    deg_kernel, edge_kernel = _sc_kernels()
    deg2 = deg_kernel(dst_p, zeros16, ones16)
    hp, dis = _scale_mm(x_p, wt_p, deg2[0], deg2[1])
    acc2 = edge_kernel(hp, src_p, dst_p, zerosDP)
    out = _epilogue(acc2[0], acc2[1], hp, dis, b_p)
    return out[:N, :OUT_DIM]
